# speedup vs baseline: 1.1375x; 1.1375x over previous
import jax
import jax.numpy as jnp
from jax.experimental import pallas as pl

_N = 128
_C = 32768
_BLOCK_ROWS = 8


def _ste_block(x_ref, o_ref):
    xb = x_ref[...]
    idx = jnp.argmax(xb, axis=1)
    iota = jax.lax.broadcasted_iota(jnp.int32, xb.shape, 1)
    o_ref[...] = (iota == idx[:, None]).astype(jnp.float32)


@jax.jit
def kernel(x):
    grid = (_N // _BLOCK_ROWS,)
    return pl.pallas_call(
        _ste_block,
        grid=grid,
        in_specs=[pl.BlockSpec((_BLOCK_ROWS, _C), lambda i: (i, 0))],
        out_specs=pl.BlockSpec((_BLOCK_ROWS, _C), lambda i: (i, 0)),
        out_shape=jax.ShapeDtypeStruct((_N, _C), jnp.float32),
    )(x)


# speedup vs baseline: 1.5722x; 1.3822x over previous
import jax
import jax.numpy as jnp
from jax.experimental import pallas as pl

_N = 128
_C = 32768
_BLOCK_ROWS = 16


def _ste_block(x_ref, o_ref):
    xb = x_ref[...]
    idx = jnp.argmax(xb, axis=1)
    iota = jax.lax.broadcasted_iota(jnp.int32, xb.shape, 1)
    o_ref[...] = (iota == idx[:, None]).astype(jnp.float32)


@jax.jit
def kernel(x):
    grid = (_N // _BLOCK_ROWS,)
    return pl.pallas_call(
        _ste_block,
        grid=grid,
        in_specs=[pl.BlockSpec((_BLOCK_ROWS, _C), lambda i: (i, 0))],
        out_specs=pl.BlockSpec((_BLOCK_ROWS, _C), lambda i: (i, 0)),
        out_shape=jax.ShapeDtypeStruct((_N, _C), jnp.float32),
    )(x)


# speedup vs baseline: 1.7017x; 1.0824x over previous
import jax
import jax.numpy as jnp
from jax.experimental import pallas as pl

_N = 128
_C = 32768
_BLOCK_ROWS = 32


def _ste_block(x_ref, o_ref):
    xb = x_ref[...]
    idx = jnp.argmax(xb, axis=1)
    iota = jax.lax.broadcasted_iota(jnp.int32, xb.shape, 1)
    o_ref[...] = (iota == idx[:, None]).astype(jnp.float32)


@jax.jit
def kernel(x):
    grid = (_N // _BLOCK_ROWS,)
    return pl.pallas_call(
        _ste_block,
        grid=grid,
        in_specs=[pl.BlockSpec((_BLOCK_ROWS, _C), lambda i: (i, 0))],
        out_specs=pl.BlockSpec((_BLOCK_ROWS, _C), lambda i: (i, 0)),
        out_shape=jax.ShapeDtypeStruct((_N, _C), jnp.float32),
    )(x)


# speedup vs baseline: 1.9275x; 1.1327x over previous
import jax
import jax.numpy as jnp
from jax.experimental import pallas as pl

_N = 128
_C = 32768
_BLOCK_ROWS = 64


def _ste_block(x_ref, o_ref):
    xb = x_ref[...]
    idx = jnp.argmax(xb, axis=1)
    iota = jax.lax.broadcasted_iota(jnp.int32, xb.shape, 1)
    o_ref[...] = (iota == idx[:, None]).astype(jnp.float32)


@jax.jit
def kernel(x):
    grid = (_N // _BLOCK_ROWS,)
    return pl.pallas_call(
        _ste_block,
        grid=grid,
        in_specs=[pl.BlockSpec((_BLOCK_ROWS, _C), lambda i: (i, 0))],
        out_specs=pl.BlockSpec((_BLOCK_ROWS, _C), lambda i: (i, 0)),
        out_shape=jax.ShapeDtypeStruct((_N, _C), jnp.float32),
    )(x)
